# Initial kernel scaffold; baseline (speedup 1.0000x reference)
import functools

import jax
import jax.numpy as jnp
from jax import lax
from jax.experimental import pallas as pl
from jax.experimental.pallas import tpu as pltpu
from jax.experimental.pallas import tpu_sc as plsc

B = 1024
D = 128
OUT = 100000
K = 512
T = 0.07

NC = 2
NS = 16
L = 16
NW = NC * NS
BPW = B // NW
CHUNK = 128
CPB = K // CHUNK
NCHUNK = BPW * CPB
XPW = BPW * D


def _sc_body(x_hbm, y_hbm, idx_hbm, mem_hbm, out_hbm, part_hbm,
             x_v, y_v, idx_v, rows0, rows1, ob0, ob1, part_v,
             sem_g0, sem_g1, sem_o0, sem_o1):
    wid = lax.axis_index("s") * NC + lax.axis_index("c")

    pltpu.sync_copy(x_hbm.at[pl.ds(wid * XPW, XPW)], x_v)
    pltpu.sync_copy(y_hbm.at[pl.ds(wid * BPW, BPW)], y_v)
    pltpu.sync_copy(idx_hbm.at[pl.ds(wid * NCHUNK, NCHUNK)], idx_v)

    inv_t = jnp.float32(1.0 / T)
    lanes = lax.broadcasted_iota(jnp.int32, (L,), 0)

    def _scale(i, carry):
        x_v[pl.ds(i * L, L)] = x_v[pl.ds(i * L, L)] * inv_t
        return carry
    lax.fori_loop(0, XPW // L, _scale, 0)

    def _patch(b, carry):
        yb = jnp.full((L,), y_v[b], jnp.int32)
        row = b * CPB
        cur = idx_v[row, pl.ds(0, L)]
        idx_v[row, pl.ds(0, L)] = jnp.where(lanes == 0, yb, cur)
        return carry
    lax.fori_loop(0, BPW, _patch, 0)

    def _issue(t, rows, sem):
        pltpu.async_copy(mem_hbm.at[idx_v.at[t]], rows, sem)

    def _gwait(t, rows, sem):
        pltpu.make_async_copy(mem_hbm.at[idx_v.at[t]], rows, sem).wait()

    def _owait(ob, sem):
        pltpu.make_async_copy(ob, out_hbm.at[0], sem).wait()

    def _compute(t, rows, ob, sem_o, sums):
        xbase = (t >> 2) * D

        def _group(g, sums):
            rowv = g * L + lanes
            acc = jnp.zeros((L,), jnp.float32)
            for jc in range(D // L):
                xv = x_v[pl.ds(xbase + jc * L, L)]
                for l in range(L):
                    xb = jnp.take(xv, jnp.full((L,), l, jnp.int32),
                                  mode="promise_in_bounds")
                    col = plsc.load_gather(
                        rows, [rowv, jnp.full((L,), jc * L + l, jnp.int32)])
                    acc = acc + col * xb
            ev = jnp.exp(acc)
            ob[pl.ds(g * L, L)] = ev
            return sums + ev

        sums = lax.fori_loop(0, CHUNK // L, _group, sums)
        pltpu.async_copy(ob, out_hbm.at[wid * NCHUNK + t], sem_o)
        return sums

    _issue(0, rows0, sem_g0)
    sums0 = jnp.zeros((L,), jnp.float32)

    def _pair(p, sums):
        t0 = 2 * p
        t1 = t0 + 1
        _issue(t1, rows1, sem_g1)
        _gwait(t0, rows0, sem_g0)

        @pl.when(t0 >= 2)
        def _():
            _owait(ob0, sem_o0)
        sums = _compute(t0, rows0, ob0, sem_o0, sums)

        @pl.when(p < NCHUNK // 2 - 1)
        def _():
            _issue(t0 + 2, rows0, sem_g0)
        _gwait(t1, rows1, sem_g1)

        @pl.when(t1 >= 2)
        def _():
            _owait(ob1, sem_o1)
        sums = _compute(t1, rows1, ob1, sem_o1, sums)
        return sums

    sums0 = lax.fori_loop(0, NCHUNK // 2, _pair, sums0)

    _owait(ob0, sem_o0)
    _owait(ob1, sem_o1)
    part_v[...] = sums0
    pltpu.sync_copy(part_v, part_hbm.at[wid])


def _norm_body(u_ref, p_ref, o_ref):
    s = jnp.sum(p_ref[...])
    scale = (jnp.float32(B) * jnp.float32(K)) / (jnp.float32(OUT) * s)
    o_ref[...] = u_ref[...] * scale


@jax.jit
def kernel(x, y, memory, idx):
    x_flat = x.reshape(B * D)
    idx_r = idx.reshape(B * CPB, CHUNK)

    mesh = plsc.VectorSubcoreMesh(core_axis_name="c", subcore_axis_name="s")
    sc_fn = pl.kernel(
        _sc_body,
        out_type=(
            jax.ShapeDtypeStruct((B * CPB, CHUNK), jnp.float32),
            jax.ShapeDtypeStruct((NW, L), jnp.float32),
        ),
        mesh=mesh,
        scratch_types=[
            pltpu.VMEM((XPW,), jnp.float32),
            pltpu.VMEM((BPW,), jnp.int32),
            pltpu.VMEM((NCHUNK, CHUNK), jnp.int32),
            pltpu.VMEM((CHUNK, D), jnp.float32),
            pltpu.VMEM((CHUNK, D), jnp.float32),
            pltpu.VMEM((CHUNK,), jnp.float32),
            pltpu.VMEM((CHUNK,), jnp.float32),
            pltpu.VMEM((L,), jnp.float32),
            pltpu.SemaphoreType.DMA,
            pltpu.SemaphoreType.DMA,
            pltpu.SemaphoreType.DMA,
            pltpu.SemaphoreType.DMA,
        ],
    )
    unnorm, part = sc_fn(x_flat, y, idx_r, memory)

    out = pl.pallas_call(
        _norm_body,
        out_shape=jax.ShapeDtypeStruct((B * CPB, CHUNK), jnp.float32),
    )(unnorm, part)
    return out.reshape(B, K)


# speedup vs baseline: 8.1094x; 8.1094x over previous
import jax
import jax.numpy as jnp
from jax import lax
from jax.experimental import pallas as pl
from jax.experimental.pallas import tpu as pltpu
from jax.experimental.pallas import tpu_sc as plsc

B = 1024
D = 128
OUT = 100000
K = 512
T = 0.07

NC = 2
NS = 16
L = 16
NW = NC * NS
BPW = B // NW
CHUNK = 128
CPB = K // CHUNK
NCHUNK = BPW * CPB
XW = BPW * D
_BITREV = (0, 8, 4, 12, 2, 10, 6, 14, 1, 9, 5, 13, 3, 11, 7, 15)


def _sc_body(x_hbm, y_hbm, idx_hbm, mem_hbm, out_hbm,
             x_v, y_v, idx_v, rows0, rows1, ob0, ob1,
             sem_g0, sem_g1, sem_o0, sem_o1):
    wid = lax.axis_index("s") * NC + lax.axis_index("c")

    pltpu.sync_copy(x_hbm.at[pl.ds(wid * XW, XW)], x_v)
    pltpu.sync_copy(y_hbm.at[pl.ds(wid * BPW, BPW)], y_v)
    pltpu.sync_copy(idx_hbm.at[pl.ds(wid * NCHUNK, NCHUNK)], idx_v)

    inv_t = jnp.float32(1.0 / T)
    lanes = lax.broadcasted_iota(jnp.int32, (L,), 0)
    splitter = jnp.float32(65537.0)

    def _bf16_round(w):
        c = w * splitter
        return c - (c - w)

    def _scale(i, carry):
        x_v[pl.ds(i * L, L)] = _bf16_round(x_v[pl.ds(i * L, L)]) * inv_t
        return carry
    lax.fori_loop(0, XW // L, _scale, 0)

    for bc in range(BPW // L):
        yv = y_v[pl.ds(bc * L, L)]
        for i in range(L):
            b = bc * L + i
            yb = yv.at[jnp.full((L,), i, jnp.int32)].get(
                mode="promise_in_bounds")
            cur = idx_v[b * CPB, pl.ds(0, L)]
            idx_v[b * CPB, pl.ds(0, L)] = jnp.where(lanes == 0, yb, cur)

    def _issue(t, rows, sem):
        pltpu.async_copy(mem_hbm.at[idx_v.at[t]], rows, sem)

    def _gwait(t, rows, sem):
        pltpu.make_async_copy(mem_hbm.at[idx_v.at[t]], rows, sem).wait()

    def _owait(ob, sem):
        pltpu.make_async_copy(ob, out_hbm.at[0], sem).wait()

    def _compute(t, rows, ob, sem_o):
        xbase = (t >> 2) * D
        xvs = [x_v[pl.ds(xbase + jc * L, L)] for jc in range(D // L)]

        def _group(g, carry):
            base = g * L
            accs = []
            for i in range(L):
                r = base + i
                acc = _bf16_round(rows[r, pl.ds(0, L)]) * xvs[0]
                for jc in range(1, D // L):
                    acc = acc + _bf16_round(rows[r, pl.ds(jc * L, L)]) * xvs[jc]
                accs.append(acc)
            accs = [accs[p] for p in _BITREV]
            for lvl in (8, 4, 2, 1):
                flip = lanes ^ lvl
                m = (lanes & lvl) == 0
                nxt = []
                for i in range(len(accs) // 2):
                    u, v = accs[2 * i], accs[2 * i + 1]
                    us = u + u.at[flip].get(mode="promise_in_bounds")
                    vs = v + v.at[flip].get(mode="promise_in_bounds")
                    nxt.append(jnp.where(m, us, vs))
                accs = nxt
            ob[pl.ds(base, L)] = accs[0]
            return carry

        lax.fori_loop(0, CHUNK // L, _group, 0)
        pltpu.async_copy(ob, out_hbm.at[wid * NCHUNK + t], sem_o)

    _issue(0, rows0, sem_g0)

    def _pair(p, carry):
        t0 = 2 * p
        t1 = t0 + 1
        _issue(t1, rows1, sem_g1)
        _gwait(t0, rows0, sem_g0)

        @pl.when(t0 >= 2)
        def _():
            _owait(ob0, sem_o0)
        _compute(t0, rows0, ob0, sem_o0)

        @pl.when(p < NCHUNK // 2 - 1)
        def _():
            _issue(t0 + 2, rows0, sem_g0)
        _gwait(t1, rows1, sem_g1)

        @pl.when(t1 >= 2)
        def _():
            _owait(ob1, sem_o1)
        _compute(t1, rows1, ob1, sem_o1)
        return carry

    lax.fori_loop(0, NCHUNK // 2, _pair, 0)

    _owait(ob0, sem_o0)
    _owait(ob1, sem_o1)


def _zsum_body(s_ref, z_ref):
    z_ref[0, 0] = jnp.sum(jnp.exp(s_ref[...]))


def _norm_body(s_ref, z_ref, o_ref):
    scale = (jnp.float32(B) * jnp.float32(K)) / (jnp.float32(OUT) * z_ref[0, 0])
    o_ref[...] = jnp.exp(s_ref[...]) * scale


@jax.jit
def kernel(x, y, memory, idx):
    x_w = x.reshape(B * D)
    idx_r = idx.reshape(B * CPB, CHUNK)

    mesh = plsc.VectorSubcoreMesh(core_axis_name="c", subcore_axis_name="s")
    sc_fn = pl.kernel(
        _sc_body,
        out_type=jax.ShapeDtypeStruct((B * CPB, CHUNK), jnp.float32),
        mesh=mesh,
        scratch_types=[
            pltpu.VMEM((XW,), jnp.float32),
            pltpu.VMEM((BPW,), jnp.int32),
            pltpu.VMEM((NCHUNK, CHUNK), jnp.int32),
            pltpu.VMEM((CHUNK, D), jnp.float32),
            pltpu.VMEM((CHUNK, D), jnp.float32),
            pltpu.VMEM((CHUNK,), jnp.float32),
            pltpu.VMEM((CHUNK,), jnp.float32),
            pltpu.SemaphoreType.DMA,
            pltpu.SemaphoreType.DMA,
            pltpu.SemaphoreType.DMA,
            pltpu.SemaphoreType.DMA,
        ],
    )
    scores = sc_fn(x_w, y, idx_r, memory)

    zsum = pl.pallas_call(
        _zsum_body,
        out_shape=jax.ShapeDtypeStruct((1, 1), jnp.float32),
        out_specs=pl.BlockSpec(memory_space=pltpu.SMEM),
    )(scores)

    out = pl.pallas_call(
        _norm_body,
        out_shape=jax.ShapeDtypeStruct((B * CPB, CHUNK), jnp.float32),
        in_specs=[
            pl.BlockSpec(memory_space=pltpu.VMEM),
            pl.BlockSpec(memory_space=pltpu.SMEM),
        ],
    )(scores, zsum)
    return out.reshape(B, K)


# speedup vs baseline: 9.1009x; 1.1223x over previous
import jax
import jax.numpy as jnp
from jax import lax
from jax.experimental import pallas as pl
from jax.experimental.pallas import tpu as pltpu
from jax.experimental.pallas import tpu_sc as plsc

B = 1024
D = 128
OUT = 100000
K = 512
T = 0.07

NC = 2
NS = 16
L = 16
NW = NC * NS
BPW = B // NW
CHUNK = 128
CPB = K // CHUNK
NCHUNK = BPW * CPB
XW = BPW * D
_BITREV = (0, 8, 4, 12, 2, 10, 6, 14, 1, 9, 5, 13, 3, 11, 7, 15)


def _sc_body(x_hbm, y_hbm, idx_hbm, mem_hbm, out_hbm,
             x_v, y_v, idx_v, rows0, rows1, ob,
             sem_g0, sem_g1):
    wid = lax.axis_index("s") * NC + lax.axis_index("c")

    pltpu.sync_copy(x_hbm.at[pl.ds(wid * XW, XW)], x_v)
    pltpu.sync_copy(y_hbm.at[pl.ds(wid * BPW, BPW)], y_v)
    pltpu.sync_copy(idx_hbm.at[pl.ds(wid * NCHUNK, NCHUNK)], idx_v)

    inv_t = jnp.float32(1.0 / T)
    lanes = lax.broadcasted_iota(jnp.int32, (L,), 0)
    splitter = jnp.float32(65537.0)

    def _bf16_round(w):
        c = w * splitter
        return c - (c - w)

    def _scale(i, carry):
        x_v[pl.ds(i * L, L)] = _bf16_round(x_v[pl.ds(i * L, L)]) * inv_t
        return carry
    lax.fori_loop(0, XW // L, _scale, 0)

    for bc in range(BPW // L):
        yv = y_v[pl.ds(bc * L, L)]
        for i in range(L):
            b = bc * L + i
            yb = yv.at[jnp.full((L,), i, jnp.int32)].get(
                mode="promise_in_bounds")
            cur = idx_v[b * CPB, pl.ds(0, L)]
            idx_v[b * CPB, pl.ds(0, L)] = jnp.where(lanes == 0, yb, cur)

    def _issue(t, rows, sem):
        pltpu.async_copy(mem_hbm.at[idx_v.at[t]], rows, sem)

    def _gwait(t, rows, sem):
        pltpu.make_async_copy(mem_hbm.at[idx_v.at[t]], rows, sem).wait()

    def _compute(t, rows):
        xbase = (t >> 2) * D
        xvs = [x_v[pl.ds(xbase + jc * L, L)] for jc in range(D // L)]

        def _group(g, carry):
            base = g * L
            accs = []
            for i in range(L):
                r = base + i
                prods = [_bf16_round(rows[r, pl.ds(jc * L, L)]) * xvs[jc]
                         for jc in range(D // L)]
                while len(prods) > 1:
                    prods = [prods[2 * i] + prods[2 * i + 1]
                             for i in range(len(prods) // 2)]
                accs.append(prods[0])
            accs = [accs[p] for p in _BITREV]
            for lvl in (8, 4, 2, 1):
                flip = lanes ^ lvl
                m = (lanes & lvl) == 0
                nxt = []
                for i in range(len(accs) // 2):
                    u, v = accs[2 * i], accs[2 * i + 1]
                    us = u + u.at[flip].get(mode="promise_in_bounds")
                    vs = v + v.at[flip].get(mode="promise_in_bounds")
                    nxt.append(jnp.where(m, us, vs))
                accs = nxt
            ob[pl.ds(t * CHUNK + base, L)] = accs[0]
            return carry

        lax.fori_loop(0, CHUNK // L, _group, 0)

    _issue(0, rows0, sem_g0)

    def _pair(p, carry):
        t0 = 2 * p
        t1 = t0 + 1
        _issue(t1, rows1, sem_g1)
        _gwait(t0, rows0, sem_g0)

        _compute(t0, rows0)

        @pl.when(p < NCHUNK // 2 - 1)
        def _():
            _issue(t0 + 2, rows0, sem_g0)
        _gwait(t1, rows1, sem_g1)

        _compute(t1, rows1)
        return carry

    lax.fori_loop(0, NCHUNK // 2, _pair, 0)

    pltpu.sync_copy(ob, out_hbm.at[pl.ds(wid * NCHUNK * CHUNK, NCHUNK * CHUNK)])


def _zsum_body(s_ref, z_ref):
    z_ref[0, 0] = jnp.sum(jnp.exp(s_ref[...]))


def _norm_body(s_ref, z_ref, o_ref):
    scale = (jnp.float32(B) * jnp.float32(K)) / (jnp.float32(OUT) * z_ref[0, 0])
    o_ref[...] = jnp.exp(s_ref[...]) * scale


@jax.jit
def kernel(x, y, memory, idx):
    x_w = x.reshape(B * D)
    idx_r = idx.reshape(B * CPB, CHUNK)

    mesh = plsc.VectorSubcoreMesh(core_axis_name="c", subcore_axis_name="s")
    sc_fn = pl.kernel(
        _sc_body,
        out_type=jax.ShapeDtypeStruct((B * K,), jnp.float32),
        mesh=mesh,
        scratch_types=[
            pltpu.VMEM((XW,), jnp.float32),
            pltpu.VMEM((BPW,), jnp.int32),
            pltpu.VMEM((NCHUNK, CHUNK), jnp.int32),
            pltpu.VMEM((CHUNK, D), jnp.float32),
            pltpu.VMEM((CHUNK, D), jnp.float32),
            pltpu.VMEM((NCHUNK * CHUNK,), jnp.float32),
            pltpu.SemaphoreType.DMA,
            pltpu.SemaphoreType.DMA,
        ],
    )
    scores = sc_fn(x_w, y, idx_r, memory).reshape(B * CPB, CHUNK)

    zsum = pl.pallas_call(
        _zsum_body,
        out_shape=jax.ShapeDtypeStruct((1, 1), jnp.float32),
        out_specs=pl.BlockSpec(memory_space=pltpu.SMEM),
    )(scores)

    out = pl.pallas_call(
        _norm_body,
        out_shape=jax.ShapeDtypeStruct((B * CPB, CHUNK), jnp.float32),
        in_specs=[
            pl.BlockSpec(memory_space=pltpu.VMEM),
            pl.BlockSpec(memory_space=pltpu.SMEM),
        ],
    )(scores, zsum)
    return out.reshape(B, K)


# speedup vs baseline: 9.1718x; 1.0078x over previous
import jax
import jax.numpy as jnp
from jax import lax
from jax.experimental import pallas as pl
from jax.experimental.pallas import tpu as pltpu
from jax.experimental.pallas import tpu_sc as plsc

B = 1024
D = 128
OUT = 100000
K = 512
T = 0.07

NC = 2
NS = 16
L = 16
NW = NC * NS
BPW = B // NW
CHUNK = 128
CPB = K // CHUNK
NCHUNK = BPW * CPB
XW = BPW * D
_BITREV = (0, 8, 4, 12, 2, 10, 6, 14, 1, 9, 5, 13, 3, 11, 7, 15)


def _sc_body(x_hbm, y_hbm, idx_hbm, mem_hbm, out_hbm,
             x_v, y_v, idx_v, rows0, rows1, ob,
             sem_g0, sem_g1):
    wid = lax.axis_index("s") * NC + lax.axis_index("c")

    pltpu.sync_copy(x_hbm.at[pl.ds(wid * XW, XW)], x_v)
    pltpu.sync_copy(y_hbm.at[pl.ds(wid * BPW, BPW)], y_v)
    pltpu.sync_copy(idx_hbm.at[pl.ds(wid * NCHUNK, NCHUNK)], idx_v)

    inv_t = jnp.float32(1.0 / T)
    lanes = lax.broadcasted_iota(jnp.int32, (L,), 0)
    splitter = jnp.float32(65537.0)

    def _bf16_round(w):
        c = w * splitter
        return c - (c - w)

    def _scale(i, carry):
        x_v[pl.ds(i * L, L)] = _bf16_round(x_v[pl.ds(i * L, L)]) * inv_t
        return carry
    lax.fori_loop(0, XW // L, _scale, 0)

    for bc in range(BPW // L):
        yv = y_v[pl.ds(bc * L, L)]
        for i in range(L):
            b = bc * L + i
            yb = yv.at[jnp.full((L,), i, jnp.int32)].get(
                mode="promise_in_bounds")
            cur = idx_v[b * CPB, pl.ds(0, L)]
            idx_v[b * CPB, pl.ds(0, L)] = jnp.where(lanes == 0, yb, cur)

    def _issue(t, rows, sem):
        pltpu.async_copy(mem_hbm.at[idx_v.at[t]], rows, sem)

    def _gwait(t, rows, sem):
        pltpu.make_async_copy(mem_hbm.at[idx_v.at[t]], rows, sem).wait()

    def _compute(t, rows):
        xbase = (t >> 2) * D
        xvs = [x_v[pl.ds(xbase + jc * L, L)] for jc in range(D // L)]

        def _group(g, carry):
            base = g * L
            accs = []
            for i in range(L):
                r = base + i
                prods = [_bf16_round(rows[r, pl.ds(jc * L, L)]) * xvs[jc]
                         for jc in range(D // L)]
                while len(prods) > 1:
                    prods = [prods[2 * i] + prods[2 * i + 1]
                             for i in range(len(prods) // 2)]
                accs.append(prods[0])
            accs = [accs[p] for p in _BITREV]
            for lvl in (8, 4, 2, 1):
                flip = lanes ^ lvl
                m = (lanes & lvl) == 0
                nxt = []
                for i in range(len(accs) // 2):
                    u, v = accs[2 * i], accs[2 * i + 1]
                    us = u + u.at[flip].get(mode="promise_in_bounds")
                    vs = v + v.at[flip].get(mode="promise_in_bounds")
                    nxt.append(jnp.where(m, us, vs))
                accs = nxt
            ob[pl.ds(t * CHUNK + base, L)] = accs[0]
            return carry

        lax.fori_loop(0, CHUNK // L, _group, 0, unroll=2)

    _issue(0, rows0, sem_g0)

    def _pair(p, carry):
        t0 = 2 * p
        t1 = t0 + 1
        _issue(t1, rows1, sem_g1)
        _gwait(t0, rows0, sem_g0)

        _compute(t0, rows0)

        @pl.when(p < NCHUNK // 2 - 1)
        def _():
            _issue(t0 + 2, rows0, sem_g0)
        _gwait(t1, rows1, sem_g1)

        _compute(t1, rows1)
        return carry

    lax.fori_loop(0, NCHUNK // 2, _pair, 0)

    pltpu.sync_copy(ob, out_hbm.at[pl.ds(wid * NCHUNK * CHUNK, NCHUNK * CHUNK)])


def _zsum_body(s_ref, z_ref):
    z_ref[0, 0] = jnp.sum(jnp.exp(s_ref[...]))


def _norm_body(s_ref, z_ref, o_ref):
    scale = (jnp.float32(B) * jnp.float32(K)) / (jnp.float32(OUT) * z_ref[0, 0])
    o_ref[...] = jnp.exp(s_ref[...]) * scale


@jax.jit
def kernel(x, y, memory, idx):
    x_w = x.reshape(B * D)
    idx_r = idx.reshape(B * CPB, CHUNK)

    mesh = plsc.VectorSubcoreMesh(core_axis_name="c", subcore_axis_name="s")
    sc_fn = pl.kernel(
        _sc_body,
        out_type=jax.ShapeDtypeStruct((B * K,), jnp.float32),
        mesh=mesh,
        scratch_types=[
            pltpu.VMEM((XW,), jnp.float32),
            pltpu.VMEM((BPW,), jnp.int32),
            pltpu.VMEM((NCHUNK, CHUNK), jnp.int32),
            pltpu.VMEM((CHUNK, D), jnp.float32),
            pltpu.VMEM((CHUNK, D), jnp.float32),
            pltpu.VMEM((NCHUNK * CHUNK,), jnp.float32),
            pltpu.SemaphoreType.DMA,
            pltpu.SemaphoreType.DMA,
        ],
    )
    scores = sc_fn(x_w, y, idx_r, memory).reshape(B * CPB, CHUNK)

    zsum = pl.pallas_call(
        _zsum_body,
        out_shape=jax.ShapeDtypeStruct((1, 1), jnp.float32),
        out_specs=pl.BlockSpec(memory_space=pltpu.SMEM),
    )(scores)

    out = pl.pallas_call(
        _norm_body,
        out_shape=jax.ShapeDtypeStruct((B * CPB, CHUNK), jnp.float32),
        in_specs=[
            pl.BlockSpec(memory_space=pltpu.VMEM),
            pl.BlockSpec(memory_space=pltpu.SMEM),
        ],
    )(scores, zsum)
    return out.reshape(B, K)
